# baseline (speedup 1.0000x reference)
import functools

import jax
import jax.numpy as jnp
from jax import lax
from jax.experimental import pallas as pl
from jax.experimental.pallas import tpu as pltpu
from jax.experimental.pallas import tpu_sc as plsc

NUM_SHARED = 8
TOP_K = 8
SC_LANES = 16
SC_WORKERS = 32
CHUNKS = 2


def _router_body(x_ref, w1_ref, b1_ref, w2_ref, b2_ref,
                 weights_ref, acc_ref, *, n1_tiles: int):
    n1 = pl.program_id(1)

    h = jnp.dot(x_ref[...], w1_ref[...],
                preferred_element_type=jnp.float32,
                precision=jax.lax.Precision.DEFAULT)
    h = h + b1_ref[...]
    h = 0.5 * h * (1.0 + jax.lax.erf(h * 0.7071067811865476))
    part = jnp.dot(h, w2_ref[...],
                   preferred_element_type=jnp.float32,
                   precision=jax.lax.Precision.DEFAULT)

    @pl.when(n1 == 0)
    def _init():
        acc_ref[...] = part

    @pl.when(n1 != 0)
    def _accum():
        acc_ref[...] += part

    @pl.when(n1 == n1_tiles - 1)
    def _finish():
        logits = acc_ref[...] + b2_ref[...]
        m = jnp.max(logits, axis=-1, keepdims=True)
        e = jnp.exp(logits - m)
        weights_ref[...] = e / jnp.sum(e, axis=-1, keepdims=True)


def _tc_router(x, W1, b1r, W2, b2r, chunk_tokens, m_off):
    input_dim = x.shape[1]
    hidden = W1.shape[1]
    num_experts = W2.shape[1]

    bm = min(512, chunk_tokens)
    bn1 = min(1024, hidden)
    m_tiles = chunk_tokens // bm
    n1_tiles = hidden // bn1
    mo = m_off // bm

    body = functools.partial(_router_body, n1_tiles=n1_tiles)

    return pl.pallas_call(
        body,
        grid=(m_tiles, n1_tiles),
        in_specs=[
            pl.BlockSpec((bm, input_dim), lambda i, j: (mo + i, 0)),
            pl.BlockSpec((input_dim, bn1), lambda i, j: (0, j)),
            pl.BlockSpec((1, bn1), lambda i, j: (0, j)),
            pl.BlockSpec((bn1, num_experts), lambda i, j: (j, 0)),
            pl.BlockSpec((1, num_experts), lambda i, j: (0, 0)),
        ],
        out_specs=pl.BlockSpec((bm, num_experts), lambda i, j: (i, 0)),
        out_shape=jax.ShapeDtypeStruct((chunk_tokens, num_experts),
                                       jnp.float32),
        scratch_shapes=[pltpu.VMEM((bm, num_experts), jnp.float32)],
        compiler_params=pltpu.CompilerParams(
            dimension_semantics=("parallel", "arbitrary"),
        ),
    )(x, W1, b1r, W2, b2r)


def _sc_topk_body(w_hbm, vals_hbm, idx_hbm, wblk, vblk, iblk,
                  *, rows_per_worker: int, tb: int):
    wid = lax.axis_index("s") * 2 + lax.axis_index("c")
    base = wid * rows_per_worker
    iota = lax.iota(jnp.int32, SC_LANES)
    g0 = iota
    g1 = iota + 16
    g2 = iota + 32
    g3 = iota + 48

    @pl.loop(0, rows_per_worker // tb)
    def _block(b):
        row0 = base + b * tb
        pltpu.sync_copy(w_hbm.at[pl.ds(row0, tb)], wblk)

        @pl.loop(0, tb)
        def _token(t):
            v0 = wblk.at[t][pl.ds(0, 16)]
            v1 = wblk.at[t][pl.ds(16, 16)]
            v2 = wblk.at[t][pl.ds(32, 16)]
            v3 = wblk.at[t][pl.ds(48, 16)]
            v0 = jnp.where(iota < NUM_SHARED, -1.0, v0)

            vals = jnp.zeros((SC_LANES,), jnp.float32)
            idxs = jnp.zeros((SC_LANES,), jnp.int32)
            for k in range(TOP_K):
                mm = jnp.maximum(jnp.maximum(v0, v1), jnp.maximum(v2, v3))
                cur = jnp.max(mm)
                cand = jnp.minimum(
                    jnp.minimum(jnp.where(v0 == cur, g0, 64),
                                jnp.where(v1 == cur, g1, 64)),
                    jnp.minimum(jnp.where(v2 == cur, g2, 64),
                                jnp.where(v3 == cur, g3, 64)))
                g = jnp.min(cand)
                vals = jnp.where(iota == k, cur, vals)
                idxs = jnp.where(iota == k, g - NUM_SHARED, idxs)
                v0 = jnp.where(g0 == g, -1.0, v0)
                v1 = jnp.where(g1 == g, -1.0, v1)
                v2 = jnp.where(g2 == g, -1.0, v2)
                v3 = jnp.where(g3 == g, -1.0, v3)
            vblk.at[t][...] = vals
            iblk.at[t][...] = idxs

        pltpu.sync_copy(vblk, vals_hbm.at[pl.ds(row0, tb)])
        pltpu.sync_copy(iblk, idx_hbm.at[pl.ds(row0, tb)])


def _sc_topk(weights):
    n_tokens = weights.shape[0]
    rows_per_worker = n_tokens // SC_WORKERS
    tb = min(64, rows_per_worker)

    mesh = plsc.VectorSubcoreMesh(core_axis_name="c", subcore_axis_name="s",
                                  num_cores=2, num_subcores=16)
    body = functools.partial(_sc_topk_body,
                             rows_per_worker=rows_per_worker, tb=tb)
    run = pl.kernel(
        body,
        out_type=[
            jax.ShapeDtypeStruct((n_tokens, SC_LANES), jnp.float32),
            jax.ShapeDtypeStruct((n_tokens, SC_LANES), jnp.int32),
        ],
        mesh=mesh,
        scratch_types=[
            pltpu.VMEM((tb, 64), jnp.float32),
            pltpu.VMEM((tb, SC_LANES), jnp.float32),
            pltpu.VMEM((tb, SC_LANES), jnp.int32),
        ],
        compiler_params=pltpu.CompilerParams(needs_layout_passes=False),
    )
    vals16, idx16 = run(weights)
    return vals16[:, :TOP_K], idx16[:, :TOP_K]


def kernel(x, W1, b1, W2, b2):
    n_tokens = x.shape[0]
    hidden = W1.shape[1]
    num_experts = W2.shape[1]

    b1r = b1.reshape(1, hidden)
    b2r = b2.reshape(1, num_experts)

    chunk = n_tokens // CHUNKS
    w_parts, v_parts, i_parts = [], [], []
    for c in range(CHUNKS):
        wc = _tc_router(x, W1, b1r, W2, b2r, chunk, c * chunk)
        vc, ic = _sc_topk(wc)
        w_parts.append(wc)
        v_parts.append(vc)
        i_parts.append(ic)

    weights = jnp.concatenate(w_parts, axis=0) if CHUNKS > 1 else w_parts[0]
    local_w = jnp.concatenate(v_parts, axis=0) if CHUNKS > 1 else v_parts[0]
    local_i = jnp.concatenate(i_parts, axis=0) if CHUNKS > 1 else i_parts[0]

    global_weights = weights[:, :NUM_SHARED]
    return (global_weights, local_w, local_i, weights)


# speedup vs baseline: 1.1416x; 1.1416x over previous
import functools

import jax
import jax.numpy as jnp
from jax import lax
from jax.experimental import pallas as pl
from jax.experimental.pallas import tpu as pltpu
from jax.experimental.pallas import tpu_sc as plsc

NUM_SHARED = 8
TOP_K = 8
SC_LANES = 16
SC_WORKERS = 32
CHUNKS = 2


def _router_body(x_ref, w1_ref, b1_ref, w2_ref, b2_ref,
                 weights_ref, acc_ref, *, n1_tiles: int):
    n1 = pl.program_id(1)

    h = jnp.dot(x_ref[...], w1_ref[...],
                preferred_element_type=jnp.float32,
                precision=jax.lax.Precision.DEFAULT)
    h = h + b1_ref[...]
    h = 0.5 * h * (1.0 + jax.lax.erf(h * 0.7071067811865476))
    part = jnp.dot(h, w2_ref[...],
                   preferred_element_type=jnp.float32,
                   precision=jax.lax.Precision.DEFAULT)

    @pl.when(n1 == 0)
    def _init():
        acc_ref[...] = part

    @pl.when(n1 != 0)
    def _accum():
        acc_ref[...] += part

    @pl.when(n1 == n1_tiles - 1)
    def _finish():
        logits = acc_ref[...] + b2_ref[...]
        m = jnp.max(logits, axis=-1, keepdims=True)
        e = jnp.exp(logits - m)
        weights_ref[...] = e / jnp.sum(e, axis=-1, keepdims=True)


def _tc_router(x, W1, b1r, W2, b2r, chunk_tokens, m_off):
    input_dim = x.shape[1]
    hidden = W1.shape[1]
    num_experts = W2.shape[1]

    bm = min(1024, chunk_tokens)
    bn1 = min(512, hidden)
    m_tiles = chunk_tokens // bm
    n1_tiles = hidden // bn1
    mo = m_off // bm

    body = functools.partial(_router_body, n1_tiles=n1_tiles)

    return pl.pallas_call(
        body,
        grid=(m_tiles, n1_tiles),
        in_specs=[
            pl.BlockSpec((bm, input_dim), lambda i, j: (mo + i, 0)),
            pl.BlockSpec((input_dim, bn1), lambda i, j: (0, j)),
            pl.BlockSpec((1, bn1), lambda i, j: (0, j)),
            pl.BlockSpec((bn1, num_experts), lambda i, j: (j, 0)),
            pl.BlockSpec((1, num_experts), lambda i, j: (0, 0)),
        ],
        out_specs=pl.BlockSpec((bm, num_experts), lambda i, j: (i, 0)),
        out_shape=jax.ShapeDtypeStruct((chunk_tokens, num_experts),
                                       jnp.float32),
        scratch_shapes=[pltpu.VMEM((bm, num_experts), jnp.float32)],
        compiler_params=pltpu.CompilerParams(
            dimension_semantics=("parallel", "arbitrary"),
        ),
    )(x, W1, b1r, W2, b2r)


def _sc_topk_body(w_hbm, vals_hbm, idx_hbm, wblk, vblk, iblk,
                  *, rows_per_worker: int, tb: int):
    wid = lax.axis_index("s") * 2 + lax.axis_index("c")
    base = wid * rows_per_worker
    iota = lax.iota(jnp.int32, SC_LANES)
    g0 = iota
    g1 = iota + 16
    g2 = iota + 32
    g3 = iota + 48

    @pl.loop(0, rows_per_worker // tb)
    def _block(b):
        row0 = base + b * tb
        pltpu.sync_copy(w_hbm.at[pl.ds(row0, tb)], wblk)

        @pl.loop(0, tb)
        def _token(t):
            v0 = wblk.at[t][pl.ds(0, 16)]
            v1 = wblk.at[t][pl.ds(16, 16)]
            v2 = wblk.at[t][pl.ds(32, 16)]
            v3 = wblk.at[t][pl.ds(48, 16)]
            v0 = jnp.where(iota < NUM_SHARED, -1.0, v0)

            vals = jnp.zeros((SC_LANES,), jnp.float32)
            idxs = jnp.zeros((SC_LANES,), jnp.int32)
            for k in range(TOP_K):
                mm = jnp.maximum(jnp.maximum(v0, v1), jnp.maximum(v2, v3))
                cur = jnp.max(mm)
                cand = jnp.minimum(
                    jnp.minimum(jnp.where(v0 == cur, g0, 64),
                                jnp.where(v1 == cur, g1, 64)),
                    jnp.minimum(jnp.where(v2 == cur, g2, 64),
                                jnp.where(v3 == cur, g3, 64)))
                g = jnp.min(cand)
                vals = jnp.where(iota == k, cur, vals)
                idxs = jnp.where(iota == k, g - NUM_SHARED, idxs)
                v0 = jnp.where(g0 == g, -1.0, v0)
                v1 = jnp.where(g1 == g, -1.0, v1)
                v2 = jnp.where(g2 == g, -1.0, v2)
                v3 = jnp.where(g3 == g, -1.0, v3)
            vblk.at[t][...] = vals
            iblk.at[t][...] = idxs

        pltpu.sync_copy(vblk, vals_hbm.at[pl.ds(row0, tb)])
        pltpu.sync_copy(iblk, idx_hbm.at[pl.ds(row0, tb)])


def _sc_topk(weights):
    n_tokens = weights.shape[0]
    rows_per_worker = n_tokens // SC_WORKERS
    tb = min(64, rows_per_worker)

    mesh = plsc.VectorSubcoreMesh(core_axis_name="c", subcore_axis_name="s",
                                  num_cores=2, num_subcores=16)
    body = functools.partial(_sc_topk_body,
                             rows_per_worker=rows_per_worker, tb=tb)
    run = pl.kernel(
        body,
        out_type=[
            jax.ShapeDtypeStruct((n_tokens, SC_LANES), jnp.float32),
            jax.ShapeDtypeStruct((n_tokens, SC_LANES), jnp.int32),
        ],
        mesh=mesh,
        scratch_types=[
            pltpu.VMEM((tb, 64), jnp.float32),
            pltpu.VMEM((tb, SC_LANES), jnp.float32),
            pltpu.VMEM((tb, SC_LANES), jnp.int32),
        ],
        compiler_params=pltpu.CompilerParams(needs_layout_passes=False),
    )
    vals16, idx16 = run(weights)
    return vals16[:, :TOP_K], idx16[:, :TOP_K]


def kernel(x, W1, b1, W2, b2):
    n_tokens = x.shape[0]
    hidden = W1.shape[1]
    num_experts = W2.shape[1]

    b1r = b1.reshape(1, hidden)
    b2r = b2.reshape(1, num_experts)

    chunk = n_tokens // CHUNKS
    w_parts, v_parts, i_parts = [], [], []
    for c in range(CHUNKS):
        wc = _tc_router(x, W1, b1r, W2, b2r, chunk, c * chunk)
        vc, ic = _sc_topk(wc)
        w_parts.append(wc)
        v_parts.append(vc)
        i_parts.append(ic)

    weights = jnp.concatenate(w_parts, axis=0) if CHUNKS > 1 else w_parts[0]
    local_w = jnp.concatenate(v_parts, axis=0) if CHUNKS > 1 else v_parts[0]
    local_i = jnp.concatenate(i_parts, axis=0) if CHUNKS > 1 else i_parts[0]

    global_weights = weights[:, :NUM_SHARED]
    return (global_weights, local_w, local_i, weights)
